# baseline (speedup 1.0000x reference)
import functools
import math

import numpy as np
import jax
import jax.numpy as jnp
from jax import lax
from jax.experimental import pallas as pl
from jax.experimental.pallas import tpu as pltpu
from jax.experimental.pallas import tpu_sc as plsc

L = 4096
R = 4096
K = L // 2 + 1
KP = 2304
KH = 2176
TOPK = 8

BM = 256
BK = 384
NKB = KP // BK

_TABLES = {}


def _dft_tables():
    if "t" in _TABLES:
        return _TABLES["t"]
    n = np.arange(L, dtype=np.int64)[:, None]
    k = np.arange(KP, dtype=np.int64)[None, :]
    m = (n * k) % L
    ph = m.astype(np.float64) * (2.0 * np.pi / L)
    valid = (k < K)
    cosf = np.where(valid, np.cos(ph), 0.0)
    sinf = np.where(valid, np.sin(ph), 0.0)
    kk = np.arange(KP, dtype=np.int64)[:, None]
    dd = np.arange(KH, dtype=np.int64)[None, :]
    phi = ((kk * dd) % L).astype(np.float64) * (2.0 * np.pi / L)
    alpha = np.where((kk == 0) | (kk == L // 2), 1.0, 2.0)
    winv = np.where((kk < K) & (dd < K), alpha * np.cos(phi) / L, 0.0)
    tri = np.triu(np.ones((16, 16), np.float32), 1)
    t = (jnp.asarray(cosf, jnp.bfloat16),
         jnp.asarray(sinf, jnp.bfloat16),
         jnp.asarray(winv, jnp.bfloat16),
         jnp.asarray(tri, jnp.float32))
    _TABLES["t"] = t
    return t


def _corr_topk_body(x_ref, c_ref, s_ref, w_ref, tri_ref, wout_ref, dout_ref,
                    acc_ref):
    kb = pl.program_id(1)
    xr = x_ref[...]
    xre = jnp.dot(xr, c_ref[...], preferred_element_type=jnp.float32)
    xim = jnp.dot(xr, s_ref[...], preferred_element_type=jnp.float32)
    p = (xre * xre + xim * xim).astype(jnp.bfloat16)
    contrib = jnp.dot(p, w_ref[...], preferred_element_type=jnp.float32)

    @pl.when(kb == 0)
    def _():
        acc_ref[...] = contrib

    @pl.when(kb > 0)
    def _():
        acc_ref[...] = acc_ref[...] + contrib

    @pl.when(kb == NKB - 1)
    def _():
        iota = lax.broadcasted_iota(jnp.int32, (BM, KH), 1)
        corr = jnp.where(iota < K, acc_ref[...], -jnp.inf)
        vals = []
        idxs = []
        for _i in range(TOPK):
            v = jnp.max(corr, axis=1, keepdims=True)
            hit = corr >= v
            ix = jnp.min(jnp.where(hit, iota, KH), axis=1, keepdims=True)
            vals.append(v)
            idxs.append(ix)
            corr = jnp.where(iota == ix, -jnp.inf, corr)
        vj = jnp.concatenate(vals, axis=1)
        dj = jnp.concatenate(idxs, axis=1)
        s16 = lax.broadcasted_iota(jnp.int32, (BM, 16), 1)
        jsl = s16 // 2
        rsl = s16 - 2 * jsl
        vE = jnp.zeros((BM, 16), jnp.float32)
        dE = jnp.zeros((BM, 16), jnp.int32)
        for jj in range(TOPK):
            vE = jnp.where(jsl == jj, vj[:, jj:jj + 1], vE)
            dE = jnp.where(jsl == jj, dj[:, jj:jj + 1], dE)
        pair_ok = (dE != 0) & (dE != L // 2)
        dE = jnp.where(rsl == 1, L - dE, dE)
        valid = (rsl == 0) | pair_ok
        validf = valid.astype(jnp.float32)
        pos = jnp.dot(validf, tri_ref[...],
                      preferred_element_type=jnp.float32).astype(jnp.int32)
        wcols = []
        dcols = []
        for sp in range(TOPK):
            m = valid & (pos == sp)
            wcols.append(jnp.sum(jnp.where(m, vE, 0.0), axis=1,
                                 keepdims=True))
            dcols.append(jnp.sum(jnp.where(m, dE, 0), axis=1,
                                 keepdims=True))
        w = jnp.concatenate(wcols, axis=1)
        d = jnp.concatenate(dcols, axis=1)
        e = jnp.exp(w - w[:, :1])
        sm = e / jnp.sum(e, axis=1, keepdims=True)
        pad = jnp.zeros((BM, 128 - TOPK), jnp.float32)
        wout_ref[...] = jnp.concatenate([sm, pad], axis=1)
        dout_ref[...] = jnp.concatenate([d, pad.astype(jnp.int32)], axis=1)


def _corr_topk(xb, cosf, sinf, winv, tri):
    grid = (R // BM, NKB)
    return pl.pallas_call(
        _corr_topk_body,
        grid=grid,
        in_specs=[
            pl.BlockSpec((BM, L), lambda i, kb: (i, 0)),
            pl.BlockSpec((L, BK), lambda i, kb: (0, kb)),
            pl.BlockSpec((L, BK), lambda i, kb: (0, kb)),
            pl.BlockSpec((BK, KH), lambda i, kb: (kb, 0)),
            pl.BlockSpec((16, 16), lambda i, kb: (0, 0)),
        ],
        out_specs=[
            pl.BlockSpec((BM, 128), lambda i, kb: (i, 0)),
            pl.BlockSpec((BM, 128), lambda i, kb: (i, 0)),
        ],
        out_shape=[
            jax.ShapeDtypeStruct((rows, 128), jnp.float32),
            jax.ShapeDtypeStruct((rows, 128), jnp.int32),
        ],
        scratch_shapes=[pltpu.VMEM((BM, KH), jnp.float32)],
        compiler_params=pltpu.CompilerParams(
            dimension_semantics=("parallel", "arbitrary"),
        ),
    )(xb, cosf, sinf, winv, tri)


NSEG = 16
NSTAGE = 4
SEGL = L // NSEG
NJ = SEGL // 2


def _bitrev4(q):
    return ((q & 1) << 3) | ((q & 2) << 1) | ((q & 4) >> 1) | ((q & 8) >> 3)


def _dft_tables_ct():
    if "ct" in _TABLES:
        return _TABLES["ct"]
    tw = np.zeros((8, L // 2), np.float32)
    for t in range(NSTAGE):
        M = L >> t
        m = np.arange(M // 2, dtype=np.float64)
        tw[t, :M // 2] = np.cos(2.0 * np.pi * m / M)
        tw[4 + t, :M // 2] = -np.sin(2.0 * np.pi * m / M)
    n = np.arange(SEGL, dtype=np.float64)[:, None]
    j = np.arange(NJ, dtype=np.float64)[None, :]
    c256 = np.cos(2.0 * np.pi * n * j / SEGL)
    s256 = np.sin(2.0 * np.pi * n * j / SEGL)
    kk = np.zeros((KH,), np.int64)
    for q in range(NSEG):
        for jj in range(NJ):
            kk[q * NJ + jj] = NSEG * jj + _bitrev4(q)
    kk[NSEG * NJ] = L // 2
    dd = np.arange(KH, dtype=np.int64)[None, :]
    phi = ((kk[:, None] * dd) % L).astype(np.float64) * (2.0 * np.pi / L)
    alpha = np.where((kk[:, None] == 0) | (kk[:, None] == L // 2), 1.0, 2.0)
    lim = NSEG * NJ + 1
    rvalid = (np.arange(KH)[:, None] < lim)
    winv = np.where(rvalid & (dd < K), alpha * np.cos(phi) / L, 0.0)
    alt = ((-1.0) ** np.arange(SEGL)).astype(np.float32)[None, :]
    tri = np.triu(np.ones((16, 16), np.float32), 1)
    t = (jnp.asarray(tw, jnp.float32),
         jnp.asarray(c256, jnp.bfloat16),
         jnp.asarray(s256, jnp.bfloat16),
         jnp.asarray(winv, jnp.bfloat16),
         jnp.asarray(alt, jnp.float32),
         jnp.asarray(tri, jnp.float32))
    _TABLES["ct"] = t
    return t


def _topk_epilogue(corr, tri_ref, wout_ref, dout_ref):
    iota = lax.broadcasted_iota(jnp.int32, (BM, KH), 1)
    corr = jnp.where(iota < K, corr, -jnp.inf)
    vals = []
    idxs = []
    for _i in range(TOPK):
        v = jnp.max(corr, axis=1, keepdims=True)
        hit = corr >= v
        ix = jnp.min(jnp.where(hit, iota, KH), axis=1, keepdims=True)
        vals.append(v)
        idxs.append(ix)
        corr = jnp.where(iota == ix, -jnp.inf, corr)
    vj = jnp.concatenate(vals, axis=1)
    dj = jnp.concatenate(idxs, axis=1)
    s16 = lax.broadcasted_iota(jnp.int32, (BM, 16), 1)
    jsl = s16 // 2
    rsl = s16 - 2 * jsl
    vE = jnp.zeros((BM, 16), jnp.float32)
    dE = jnp.zeros((BM, 16), jnp.int32)
    for jj in range(TOPK):
        vE = jnp.where(jsl == jj, vj[:, jj:jj + 1], vE)
        dE = jnp.where(jsl == jj, dj[:, jj:jj + 1], dE)
    pair_ok = (dE != 0) & (dE != L // 2)
    dE = jnp.where(rsl == 1, L - dE, dE)
    valid = (rsl == 0) | pair_ok
    validf = valid.astype(jnp.float32)
    pos = jnp.dot(validf, tri_ref[...],
                  preferred_element_type=jnp.float32).astype(jnp.int32)
    wcols = []
    dcols = []
    for sp in range(TOPK):
        m = valid & (pos == sp)
        wcols.append(jnp.sum(jnp.where(m, vE, 0.0), axis=1, keepdims=True))
        dcols.append(jnp.sum(jnp.where(m, dE, 0), axis=1, keepdims=True))
    w = jnp.concatenate(wcols, axis=1)
    d = jnp.concatenate(dcols, axis=1)
    e = jnp.exp(w - w[:, :1])
    sm = e / jnp.sum(e, axis=1, keepdims=True)
    pad = jnp.zeros((BM, 128 - TOPK), jnp.float32)
    wout_ref[...] = jnp.concatenate([sm, pad], axis=1)
    dout_ref[...] = jnp.concatenate([d, pad.astype(jnp.int32)], axis=1)


def _corr_topk_ct_body(x_ref, tw_ref, c_ref, s_ref, w_ref, alt_ref, tri_ref,
                       wout_ref, dout_ref, zr, zi, pp, cbuf):
    i = pl.program_id(0)
    phase = i % 2
    corr_prev = cbuf[pl.ds((1 - phase) * BM, BM), :]
    H = L // 2
    ar = x_ref[:, :H]
    br = x_ref[:, H:]
    twr = tw_ref[0:1, :H]
    twi = tw_ref[4:5, :H]
    zr[:, :H] = ar + br
    zi[:, :H] = jnp.zeros((BM, H), jnp.float32)
    dr = ar - br
    zr[:, H:] = dr * twr
    zi[:, H:] = dr * twi
    for t in range(1, NSTAGE):
        M = L >> t
        H = M // 2
        twr = tw_ref[t:t + 1, :H]
        twi = tw_ref[4 + t:5 + t, :H]
        for s in range(1 << t):
            s0 = s * M
            ar = zr[:, s0:s0 + H]
            br = zr[:, s0 + H:s0 + M]
            ai = zi[:, s0:s0 + H]
            bi = zi[:, s0 + H:s0 + M]
            zr[:, s0:s0 + H] = ar + br
            zi[:, s0:s0 + H] = ai + bi
            dr = ar - br
            di = ai - bi
            zr[:, s0 + H:s0 + M] = dr * twr - di * twi
            zi[:, s0 + H:s0 + M] = dr * twi + di * twr
    c256 = c_ref[...]
    s256 = s_ref[...]
    for q in range(NSEG):
        q0 = q * SEGL
        sr = zr[:, q0:q0 + SEGL].astype(jnp.bfloat16)
        si = zi[:, q0:q0 + SEGL].astype(jnp.bfloat16)
        yr = (jnp.dot(sr, c256, preferred_element_type=jnp.float32)
              + jnp.dot(si, s256, preferred_element_type=jnp.float32))
        yi = (jnp.dot(si, c256, preferred_element_type=jnp.float32)
              - jnp.dot(sr, s256, preferred_element_type=jnp.float32))
        pp[:, q * NJ:(q + 1) * NJ] = yr * yr + yi * yi
    alt = alt_ref[...]
    nr = jnp.sum(zr[:, :SEGL] * alt, axis=1, keepdims=True)
    ni = jnp.sum(zi[:, :SEGL] * alt, axis=1, keepdims=True)
    pnyq = nr * nr + ni * ni
    zpad = jnp.zeros((BM, NJ - 1), jnp.float32)
    pp[:, NSEG * NJ:] = jnp.concatenate([pnyq, zpad], axis=1)
    corr = jnp.dot(pp[...].astype(jnp.bfloat16), w_ref[...],
                   preferred_element_type=jnp.float32)
    cbuf[pl.ds(phase * BM, BM), :] = corr
    _topk_epilogue(corr_prev, tri_ref, wout_ref, dout_ref)


NBLK = R // BM


def _corr_topk_ct(x2, tw, c256, s256, winv, alt, tri):
    rows = x2.shape[0]
    nblk = rows // BM
    grid = (nblk + 1,)
    return pl.pallas_call(
        _corr_topk_ct_body,
        grid=grid,
        in_specs=[
            pl.BlockSpec((BM, L), lambda i: (jnp.minimum(i, nblk - 1), 0)),
            pl.BlockSpec((8, L // 2), lambda i: (0, 0)),
            pl.BlockSpec((SEGL, NJ), lambda i: (0, 0)),
            pl.BlockSpec((SEGL, NJ), lambda i: (0, 0)),
            pl.BlockSpec((KH, KH), lambda i: (0, 0)),
            pl.BlockSpec((1, SEGL), lambda i: (0, 0)),
            pl.BlockSpec((16, 16), lambda i: (0, 0)),
        ],
        out_specs=[
            pl.BlockSpec((BM, 128), lambda i: (jnp.maximum(i - 1, 0), 0)),
            pl.BlockSpec((BM, 128), lambda i: (jnp.maximum(i - 1, 0), 0)),
        ],
        out_shape=[
            jax.ShapeDtypeStruct((rows, 128), jnp.float32),
            jax.ShapeDtypeStruct((rows, 128), jnp.int32),
        ],
        scratch_shapes=[
            pltpu.VMEM((BM, L), jnp.float32),
            pltpu.VMEM((BM, L), jnp.float32),
            pltpu.VMEM((BM, KH), jnp.float32),
            pltpu.VMEM((2 * BM, KH), jnp.float32),
        ],
        compiler_params=pltpu.CompilerParams(
            dimension_semantics=("arbitrary",),
        ),
    )(x2, tw, c256, s256, winv, alt, tri)


NWORKERS = 32
ROWS_PER_W = R // NWORKERS


def _make_agg_body(rows_per):
  def _agg_body(x_hbm, w_hbm, d_hbm, out_hbm, xbuf0, xbuf1, wall, dall,
                obuf0, obuf1, sin0, sin1, so0, so1):
    cid = lax.axis_index("c")
    sid = lax.axis_index("s")
    wid = sid * 2 + cid
    lanes = lax.iota(jnp.int32, 16)

    base = wid * rows_per

    pltpu.sync_copy(w_hbm.at[pl.ds(base * 16, rows_per * 16)], wall)
    pltpu.sync_copy(d_hbm.at[pl.ds(base * 16, rows_per * 16)], dall)

    xbufs = (xbuf0, xbuf1)
    obufs = (obuf0, obuf1)
    sems_in = (sin0, sin1)
    sems_out = (so0, so1)
    pltpu.async_copy(x_hbm.at[base], xbuf0, sin0)

    def process(r, b):
        nb = 1 - b

        @pl.when(r + 1 < rows_per)
        def _():
            pltpu.async_copy(x_hbm.at[base + r + 1], xbufs[nb],
                             sems_in[nb])

        wvec = plsc.load_gather(wall, (r * 16 + lanes,))
        dvec = plsc.load_gather(dall, (r * 16 + lanes,))
        sel0 = lanes == 0
        db0 = jnp.sum(jnp.where(sel0, dvec, 0))
        wb0 = jnp.sum(jnp.where(sel0, wvec, jnp.float32(0)))
        tailsel = (lanes >= 1) & (lanes < TOPK)
        wtail = jnp.sum(jnp.where(tailsel, wvec, jnp.float32(0)))

        pltpu.make_async_copy(x_hbm.at[base + r], xbufs[b],
                              sems_in[b]).wait()

        @pl.when(r >= 2)
        def _():
            pltpu.make_async_copy(obufs[b], out_hbm.at[base + r - 2],
                                  sems_out[b]).wait()

        @pl.when(db0 == 0)
        def _():
            def chunk0l(j, c2):
                sl = pl.ds(j * 16, 16)
                obufs[b][sl] = wb0 * xbufs[b][sl]
                return c2

            lax.fori_loop(0, L // 16, chunk0l, 0, unroll=4)

        @pl.when(db0 != 0)
        def _():
            def chunk0(j, c2):
                idx = (j * 16 + lanes + db0) & (L - 1)
                obufs[b][pl.ds(j * 16, 16)] = wb0 * plsc.load_gather(
                    xbufs[b], (idx,))
                return c2

            lax.fori_loop(0, L // 16, chunk0, 0, unroll=4)

        @pl.when(wtail != 0.0)
        def _():
            for i in range(1, TOPK):
                sel = lanes == i
                db = jnp.sum(jnp.where(sel, dvec, 0))
                wb = jnp.sum(jnp.where(sel, wvec, jnp.float32(0)))

                @pl.when(wb != 0.0)
                def _(db=db, wb=wb):
                    def chunk(j, c2):
                        idx = (j * 16 + lanes + db) & (L - 1)
                        sl = pl.ds(j * 16, 16)
                        obufs[b][sl] = obufs[b][sl] + wb * plsc.load_gather(
                            xbufs[b], (idx,))
                        return c2

                    lax.fori_loop(0, L // 16, chunk, 0, unroll=4)

        pltpu.async_copy(obufs[b], out_hbm.at[base + r], sems_out[b])

    def outer(gg, carry):
        process(2 * gg, 0)
        process(2 * gg + 1, 1)
        return carry

    lax.fori_loop(0, rows_per // 2, outer, 0)
    pltpu.make_async_copy(obuf0, out_hbm.at[base + rows_per - 2],
                          so0).wait()
    pltpu.make_async_copy(obuf1, out_hbm.at[base + rows_per - 1],
                          so1).wait()
  return _agg_body


_AGG = {}


def _aggregate_kernel(rows):
    if rows in _AGG:
        return _AGG[rows]
    rows_per = rows // NWORKERS
    k = pl.kernel(
        _make_agg_body(rows_per),
        out_type=jax.ShapeDtypeStruct((rows, L), jnp.float32),
        mesh=plsc.VectorSubcoreMesh(
            core_axis_name="c", subcore_axis_name="s",
            num_cores=2, num_subcores=16,
        ),
        scratch_types=[
            pltpu.VMEM((L,), jnp.float32),
            pltpu.VMEM((L,), jnp.float32),
            pltpu.VMEM((rows_per * 16,), jnp.float32),
            pltpu.VMEM((rows_per * 16,), jnp.int32),
            pltpu.VMEM((L,), jnp.float32),
            pltpu.VMEM((L,), jnp.float32),
            pltpu.SemaphoreType.DMA,
            pltpu.SemaphoreType.DMA,
            pltpu.SemaphoreType.DMA,
            pltpu.SemaphoreType.DMA,
        ],
        compiler_params=pltpu.CompilerParams(needs_layout_passes=False),
    )
    _AGG[rows] = k
    return k


NSPLIT = 2


def kernel(x):
    B, C, _ = x.shape
    x2 = x.reshape(R, L)
    tw, c256, s256, winv, alt, tri = _dft_tables_ct()
    rows_h = R // NSPLIT
    outs = []
    for h in range(NSPLIT):
        xh = lax.slice(x2, (h * rows_h, 0), ((h + 1) * rows_h, L))
        wpad, dpad = _corr_topk_ct(xh, tw, c256, s256, winv, alt, tri)
        w16 = wpad[:, :16].reshape(-1)
        d16 = dpad[:, :16].reshape(-1)
        outs.append(_aggregate_kernel(rows_h)(xh, w16, d16))
    out = jnp.concatenate(outs, axis=0)
    return out.reshape(B, C, L)


# speedup vs baseline: 1.3589x; 1.3589x over previous
import functools
import math

import numpy as np
import jax
import jax.numpy as jnp
from jax import lax
from jax.experimental import pallas as pl
from jax.experimental.pallas import tpu as pltpu
from jax.experimental.pallas import tpu_sc as plsc

L = 4096
R = 4096
K = L // 2 + 1
KP = 2304
KH = 2176
TOPK = 8

BM = 256
BK = 384
NKB = KP // BK

_TABLES = {}


def _dft_tables():
    if "t" in _TABLES:
        return _TABLES["t"]
    n = np.arange(L, dtype=np.int64)[:, None]
    k = np.arange(KP, dtype=np.int64)[None, :]
    m = (n * k) % L
    ph = m.astype(np.float64) * (2.0 * np.pi / L)
    valid = (k < K)
    cosf = np.where(valid, np.cos(ph), 0.0)
    sinf = np.where(valid, np.sin(ph), 0.0)
    kk = np.arange(KP, dtype=np.int64)[:, None]
    dd = np.arange(KH, dtype=np.int64)[None, :]
    phi = ((kk * dd) % L).astype(np.float64) * (2.0 * np.pi / L)
    alpha = np.where((kk == 0) | (kk == L // 2), 1.0, 2.0)
    winv = np.where((kk < K) & (dd < K), alpha * np.cos(phi) / L, 0.0)
    tri = np.triu(np.ones((16, 16), np.float32), 1)
    t = (jnp.asarray(cosf, jnp.bfloat16),
         jnp.asarray(sinf, jnp.bfloat16),
         jnp.asarray(winv, jnp.bfloat16),
         jnp.asarray(tri, jnp.float32))
    _TABLES["t"] = t
    return t


def _corr_topk_body(x_ref, c_ref, s_ref, w_ref, tri_ref, wout_ref, dout_ref,
                    acc_ref):
    kb = pl.program_id(1)
    xr = x_ref[...]
    xre = jnp.dot(xr, c_ref[...], preferred_element_type=jnp.float32)
    xim = jnp.dot(xr, s_ref[...], preferred_element_type=jnp.float32)
    p = (xre * xre + xim * xim).astype(jnp.bfloat16)
    contrib = jnp.dot(p, w_ref[...], preferred_element_type=jnp.float32)

    @pl.when(kb == 0)
    def _():
        acc_ref[...] = contrib

    @pl.when(kb > 0)
    def _():
        acc_ref[...] = acc_ref[...] + contrib

    @pl.when(kb == NKB - 1)
    def _():
        iota = lax.broadcasted_iota(jnp.int32, (BM, KH), 1)
        corr = jnp.where(iota < K, acc_ref[...], -jnp.inf)
        vals = []
        idxs = []
        for _i in range(TOPK):
            v = jnp.max(corr, axis=1, keepdims=True)
            hit = corr >= v
            ix = jnp.min(jnp.where(hit, iota, KH), axis=1, keepdims=True)
            vals.append(v)
            idxs.append(ix)
            corr = jnp.where(iota == ix, -jnp.inf, corr)
        vj = jnp.concatenate(vals, axis=1)
        dj = jnp.concatenate(idxs, axis=1)
        s16 = lax.broadcasted_iota(jnp.int32, (BM, 16), 1)
        jsl = s16 // 2
        rsl = s16 - 2 * jsl
        vE = jnp.zeros((BM, 16), jnp.float32)
        dE = jnp.zeros((BM, 16), jnp.int32)
        for jj in range(TOPK):
            vE = jnp.where(jsl == jj, vj[:, jj:jj + 1], vE)
            dE = jnp.where(jsl == jj, dj[:, jj:jj + 1], dE)
        pair_ok = (dE != 0) & (dE != L // 2)
        dE = jnp.where(rsl == 1, L - dE, dE)
        valid = (rsl == 0) | pair_ok
        validf = valid.astype(jnp.float32)
        pos = jnp.dot(validf, tri_ref[...],
                      preferred_element_type=jnp.float32).astype(jnp.int32)
        wcols = []
        dcols = []
        for sp in range(TOPK):
            m = valid & (pos == sp)
            wcols.append(jnp.sum(jnp.where(m, vE, 0.0), axis=1,
                                 keepdims=True))
            dcols.append(jnp.sum(jnp.where(m, dE, 0), axis=1,
                                 keepdims=True))
        w = jnp.concatenate(wcols, axis=1)
        d = jnp.concatenate(dcols, axis=1)
        e = jnp.exp(w - w[:, :1])
        sm = e / jnp.sum(e, axis=1, keepdims=True)
        pad = jnp.zeros((BM, 128 - TOPK), jnp.float32)
        wout_ref[...] = jnp.concatenate([sm, pad], axis=1)
        dout_ref[...] = jnp.concatenate([d, pad.astype(jnp.int32)], axis=1)


def _corr_topk(xb, cosf, sinf, winv, tri):
    grid = (R // BM, NKB)
    return pl.pallas_call(
        _corr_topk_body,
        grid=grid,
        in_specs=[
            pl.BlockSpec((BM, L), lambda i, kb: (i, 0)),
            pl.BlockSpec((L, BK), lambda i, kb: (0, kb)),
            pl.BlockSpec((L, BK), lambda i, kb: (0, kb)),
            pl.BlockSpec((BK, KH), lambda i, kb: (kb, 0)),
            pl.BlockSpec((16, 16), lambda i, kb: (0, 0)),
        ],
        out_specs=[
            pl.BlockSpec((BM, 128), lambda i, kb: (i, 0)),
            pl.BlockSpec((BM, 128), lambda i, kb: (i, 0)),
        ],
        out_shape=[
            jax.ShapeDtypeStruct((R, 128), jnp.float32),
            jax.ShapeDtypeStruct((R, 128), jnp.int32),
        ],
        scratch_shapes=[pltpu.VMEM((BM, KH), jnp.float32)],
        compiler_params=pltpu.CompilerParams(
            dimension_semantics=("parallel", "arbitrary"),
        ),
    )(xb, cosf, sinf, winv, tri)


NSEG = 16
NSTAGE = 4
SEGL = L // NSEG
NJ = SEGL // 2


def _bitrev4(q):
    return ((q & 1) << 3) | ((q & 2) << 1) | ((q & 4) >> 1) | ((q & 8) >> 3)


def _dft_tables_ct():
    if "ct" in _TABLES:
        return _TABLES["ct"]
    tw = np.zeros((8, L // 2), np.float32)
    for t in range(NSTAGE):
        M = L >> t
        m = np.arange(M // 2, dtype=np.float64)
        tw[t, :M // 2] = np.cos(2.0 * np.pi * m / M)
        tw[4 + t, :M // 2] = -np.sin(2.0 * np.pi * m / M)
    n = np.arange(SEGL, dtype=np.float64)[:, None]
    j = np.arange(NJ, dtype=np.float64)[None, :]
    c256 = np.cos(2.0 * np.pi * n * j / SEGL)
    s256 = np.sin(2.0 * np.pi * n * j / SEGL)
    kk = np.zeros((KH,), np.int64)
    for q in range(NSEG):
        for jj in range(NJ):
            kk[q * NJ + jj] = NSEG * jj + _bitrev4(q)
    kk[NSEG * NJ] = L // 2
    dd = np.arange(KH, dtype=np.int64)[None, :]
    phi = ((kk[:, None] * dd) % L).astype(np.float64) * (2.0 * np.pi / L)
    alpha = np.where((kk[:, None] == 0) | (kk[:, None] == L // 2), 1.0, 2.0)
    lim = NSEG * NJ + 1
    rvalid = (np.arange(KH)[:, None] < lim)
    winv = np.where(rvalid & (dd < K), alpha * np.cos(phi) / L, 0.0)
    alt = ((-1.0) ** np.arange(SEGL)).astype(np.float32)[None, :]
    tri = np.triu(np.ones((16, 16), np.float32), 1)
    t = (jnp.asarray(tw, jnp.float32),
         jnp.asarray(c256, jnp.bfloat16),
         jnp.asarray(s256, jnp.bfloat16),
         jnp.asarray(winv, jnp.bfloat16),
         jnp.asarray(alt, jnp.float32),
         jnp.asarray(tri, jnp.float32))
    _TABLES["ct"] = t
    return t


def _topk_epilogue(corr, tri_ref, wout_ref, dout_ref):
    iota = lax.broadcasted_iota(jnp.int32, (BM, KH), 1)
    corr = jnp.where(iota < K, corr, -jnp.inf)
    vals = []
    idxs = []
    for _i in range(TOPK):
        v = jnp.max(corr, axis=1, keepdims=True)
        hit = corr >= v
        ix = jnp.min(jnp.where(hit, iota, KH), axis=1, keepdims=True)
        vals.append(v)
        idxs.append(ix)
        corr = jnp.where(iota == ix, -jnp.inf, corr)
    vj = jnp.concatenate(vals, axis=1)
    dj = jnp.concatenate(idxs, axis=1)
    s16 = lax.broadcasted_iota(jnp.int32, (BM, 16), 1)
    jsl = s16 // 2
    rsl = s16 - 2 * jsl
    vE = jnp.zeros((BM, 16), jnp.float32)
    dE = jnp.zeros((BM, 16), jnp.int32)
    for jj in range(TOPK):
        vE = jnp.where(jsl == jj, vj[:, jj:jj + 1], vE)
        dE = jnp.where(jsl == jj, dj[:, jj:jj + 1], dE)
    pair_ok = (dE != 0) & (dE != L // 2)
    dE = jnp.where(rsl == 1, L - dE, dE)
    valid = (rsl == 0) | pair_ok
    validf = valid.astype(jnp.float32)
    pos = jnp.dot(validf, tri_ref[...],
                  preferred_element_type=jnp.float32).astype(jnp.int32)
    wcols = []
    dcols = []
    for sp in range(TOPK):
        m = valid & (pos == sp)
        wcols.append(jnp.sum(jnp.where(m, vE, 0.0), axis=1, keepdims=True))
        dcols.append(jnp.sum(jnp.where(m, dE, 0), axis=1, keepdims=True))
    w = jnp.concatenate(wcols, axis=1)
    d = jnp.concatenate(dcols, axis=1)
    e = jnp.exp(w - w[:, :1])
    sm = e / jnp.sum(e, axis=1, keepdims=True)
    pad = jnp.zeros((BM, 128 - TOPK), jnp.float32)
    wout_ref[...] = jnp.concatenate([sm, pad], axis=1)
    dout_ref[...] = jnp.concatenate([d, pad.astype(jnp.int32)], axis=1)


def _corr_topk_ct_body(x_ref, tw_ref, c_ref, s_ref, w_ref, alt_ref, tri_ref,
                       wout_ref, dout_ref, zr, zi, pp, cbuf):
    i = pl.program_id(0)
    phase = i % 2
    corr_prev = cbuf[pl.ds((1 - phase) * BM, BM), :]
    H = L // 2
    ar = x_ref[:, :H]
    br = x_ref[:, H:]
    twr = tw_ref[0:1, :H]
    twi = tw_ref[4:5, :H]
    zr[:, :H] = ar + br
    zi[:, :H] = jnp.zeros((BM, H), jnp.float32)
    dr = ar - br
    zr[:, H:] = dr * twr
    zi[:, H:] = dr * twi
    for t in range(1, NSTAGE):
        M = L >> t
        H = M // 2
        twr = tw_ref[t:t + 1, :H]
        twi = tw_ref[4 + t:5 + t, :H]
        for s in range(1 << t):
            s0 = s * M
            ar = zr[:, s0:s0 + H]
            br = zr[:, s0 + H:s0 + M]
            ai = zi[:, s0:s0 + H]
            bi = zi[:, s0 + H:s0 + M]
            zr[:, s0:s0 + H] = ar + br
            zi[:, s0:s0 + H] = ai + bi
            dr = ar - br
            di = ai - bi
            zr[:, s0 + H:s0 + M] = dr * twr - di * twi
            zi[:, s0 + H:s0 + M] = dr * twi + di * twr
    c256 = c_ref[...]
    s256 = s_ref[...]
    for q in range(NSEG):
        q0 = q * SEGL
        sr = zr[:, q0:q0 + SEGL].astype(jnp.bfloat16)
        si = zi[:, q0:q0 + SEGL].astype(jnp.bfloat16)
        yr = (jnp.dot(sr, c256, preferred_element_type=jnp.float32)
              + jnp.dot(si, s256, preferred_element_type=jnp.float32))
        yi = (jnp.dot(si, c256, preferred_element_type=jnp.float32)
              - jnp.dot(sr, s256, preferred_element_type=jnp.float32))
        pp[:, q * NJ:(q + 1) * NJ] = yr * yr + yi * yi
    alt = alt_ref[...]
    nr = jnp.sum(zr[:, :SEGL] * alt, axis=1, keepdims=True)
    ni = jnp.sum(zi[:, :SEGL] * alt, axis=1, keepdims=True)
    pnyq = nr * nr + ni * ni
    zpad = jnp.zeros((BM, NJ - 1), jnp.float32)
    pp[:, NSEG * NJ:] = jnp.concatenate([pnyq, zpad], axis=1)
    corr = jnp.dot(pp[...].astype(jnp.bfloat16), w_ref[...],
                   preferred_element_type=jnp.float32)
    cbuf[pl.ds(phase * BM, BM), :] = corr
    _topk_epilogue(corr_prev, tri_ref, wout_ref, dout_ref)


NBLK = R // BM


def _corr_topk_ct(x2, tw, c256, s256, winv, alt, tri):
    grid = (NBLK + 1,)
    return pl.pallas_call(
        _corr_topk_ct_body,
        grid=grid,
        in_specs=[
            pl.BlockSpec((BM, L), lambda i: (jnp.minimum(i, NBLK - 1), 0)),
            pl.BlockSpec((8, L // 2), lambda i: (0, 0)),
            pl.BlockSpec((SEGL, NJ), lambda i: (0, 0)),
            pl.BlockSpec((SEGL, NJ), lambda i: (0, 0)),
            pl.BlockSpec((KH, KH), lambda i: (0, 0)),
            pl.BlockSpec((1, SEGL), lambda i: (0, 0)),
            pl.BlockSpec((16, 16), lambda i: (0, 0)),
        ],
        out_specs=[
            pl.BlockSpec((BM, 128), lambda i: (jnp.maximum(i - 1, 0), 0)),
            pl.BlockSpec((BM, 128), lambda i: (jnp.maximum(i - 1, 0), 0)),
        ],
        out_shape=[
            jax.ShapeDtypeStruct((R, 128), jnp.float32),
            jax.ShapeDtypeStruct((R, 128), jnp.int32),
        ],
        scratch_shapes=[
            pltpu.VMEM((BM, L), jnp.float32),
            pltpu.VMEM((BM, L), jnp.float32),
            pltpu.VMEM((BM, KH), jnp.float32),
            pltpu.VMEM((2 * BM, KH), jnp.float32),
        ],
        compiler_params=pltpu.CompilerParams(
            dimension_semantics=("arbitrary",),
        ),
    )(x2, tw, c256, s256, winv, alt, tri)


NWORKERS = 32
ROWS_PER_W = R // NWORKERS


def _agg_body(x_hbm, w_hbm, d_hbm, out_hbm, xbuf0, xbuf1, wall, dall,
              obuf0, obuf1, sin0, sin1, so0, so1):
    cid = lax.axis_index("c")
    sid = lax.axis_index("s")
    wid = sid * 2 + cid
    lanes = lax.iota(jnp.int32, 16)

    base = wid * ROWS_PER_W

    pltpu.sync_copy(w_hbm.at[pl.ds(base * 16, ROWS_PER_W * 16)], wall)
    pltpu.sync_copy(d_hbm.at[pl.ds(base * 16, ROWS_PER_W * 16)], dall)

    xbufs = (xbuf0, xbuf1)
    obufs = (obuf0, obuf1)
    sems_in = (sin0, sin1)
    sems_out = (so0, so1)
    pltpu.async_copy(x_hbm.at[base], xbuf0, sin0)

    def process(r, b):
        nb = 1 - b

        @pl.when(r + 1 < ROWS_PER_W)
        def _():
            pltpu.async_copy(x_hbm.at[base + r + 1], xbufs[nb],
                             sems_in[nb])

        wvec = plsc.load_gather(wall, (r * 16 + lanes,))
        dvec = plsc.load_gather(dall, (r * 16 + lanes,))
        sel0 = lanes == 0
        db0 = jnp.sum(jnp.where(sel0, dvec, 0))
        wb0 = jnp.sum(jnp.where(sel0, wvec, jnp.float32(0)))
        tailsel = (lanes >= 1) & (lanes < TOPK)
        wtail = jnp.sum(jnp.where(tailsel, wvec, jnp.float32(0)))

        pltpu.make_async_copy(x_hbm.at[base + r], xbufs[b],
                              sems_in[b]).wait()

        @pl.when(r >= 2)
        def _():
            pltpu.make_async_copy(obufs[b], out_hbm.at[base + r - 2],
                                  sems_out[b]).wait()

        @pl.when(db0 == 0)
        def _():
            @plsc.parallel_loop(0, L // 16, unroll=8)
            def chunk0l(j):
                sl = pl.ds(j * 16, 16)
                obufs[b][sl] = wb0 * xbufs[b][sl]

        @pl.when(db0 != 0)
        def _():
            @plsc.parallel_loop(0, L // 16, unroll=8)
            def chunk0(j):
                idx = (j * 16 + lanes + db0) & (L - 1)
                obufs[b][pl.ds(j * 16, 16)] = wb0 * plsc.load_gather(
                    xbufs[b], (idx,))

        @pl.when(wtail != 0.0)
        def _():
            for i in range(1, TOPK):
                sel = lanes == i
                db = jnp.sum(jnp.where(sel, dvec, 0))
                wb = jnp.sum(jnp.where(sel, wvec, jnp.float32(0)))

                @pl.when(wb != 0.0)
                def _(db=db, wb=wb):
                    @plsc.parallel_loop(0, L // 16, unroll=8)
                    def chunk(j):
                        idx = (j * 16 + lanes + db) & (L - 1)
                        sl = pl.ds(j * 16, 16)
                        obufs[b][sl] = obufs[b][sl] + wb * plsc.load_gather(
                            xbufs[b], (idx,))

        pltpu.async_copy(obufs[b], out_hbm.at[base + r], sems_out[b])

    def outer(gg, carry):
        process(2 * gg, 0)
        process(2 * gg + 1, 1)
        return carry

    lax.fori_loop(0, ROWS_PER_W // 2, outer, 0)
    pltpu.make_async_copy(obuf0, out_hbm.at[base + ROWS_PER_W - 2],
                          so0).wait()
    pltpu.make_async_copy(obuf1, out_hbm.at[base + ROWS_PER_W - 1],
                          so1).wait()


_AGG = {}


def _aggregate_kernel():
    if "k" in _AGG:
        return _AGG["k"]
    k = pl.kernel(
        _agg_body,
        out_type=jax.ShapeDtypeStruct((R, L), jnp.float32),
        mesh=plsc.VectorSubcoreMesh(
            core_axis_name="c", subcore_axis_name="s",
            num_cores=2, num_subcores=16,
        ),
        scratch_types=[
            pltpu.VMEM((L,), jnp.float32),
            pltpu.VMEM((L,), jnp.float32),
            pltpu.VMEM((ROWS_PER_W * 16,), jnp.float32),
            pltpu.VMEM((ROWS_PER_W * 16,), jnp.int32),
            pltpu.VMEM((L,), jnp.float32),
            pltpu.VMEM((L,), jnp.float32),
            pltpu.SemaphoreType.DMA,
            pltpu.SemaphoreType.DMA,
            pltpu.SemaphoreType.DMA,
            pltpu.SemaphoreType.DMA,
        ],
        compiler_params=pltpu.CompilerParams(needs_layout_passes=False),
    )
    _AGG["k"] = k
    return k


def kernel(x):
    B, C, _ = x.shape
    x2 = x.reshape(R, L)
    tw, c256, s256, winv, alt, tri = _dft_tables_ct()
    wpad, dpad = _corr_topk_ct(x2, tw, c256, s256, winv, alt, tri)
    w16 = wpad[:, :16].reshape(-1)
    d16 = dpad[:, :16].reshape(-1)
    out = _aggregate_kernel()(x2, w16, d16)
    return out.reshape(B, C, L)
